# baseline (speedup 1.0000x reference)
import functools

import jax
import jax.numpy as jnp
from jax import lax
from jax.experimental import pallas as pl
from jax.experimental.pallas import tpu as pltpu
from jax.experimental.pallas import tpu_sc as plsc

NC = 2
NS = 16
CH = 128
GRP = 8


def _make_mm_body(n_real, blk):
    def _mm_body(x_ref, w_ref, o_ref):
        i = pl.program_id(0)
        h = jnp.dot(x_ref[...], w_ref[...],
                    preferred_element_type=jnp.float32)
        rows = i * blk + lax.broadcasted_iota(jnp.int32, h.shape, 0)
        o_ref[...] = jnp.where(rows < n_real, h, 0.0)
    return _mm_body


def _make_out_body(c_real):
    def _out_body(s2_ref, g_ref, dinv_ref, w_ref, be_ref, o_ref):
        pre = dinv_ref[...] * (s2_ref[0] + s2_ref[1] + g_ref[...])
        agg = jnp.dot(pre, w_ref[...],
                      preferred_element_type=jnp.float32) + be_ref[...]
        col = lax.broadcasted_iota(jnp.int32, agg.shape, 1)
        xm = jnp.where(col < c_real, agg, jnp.float32(-1e30))
        m = jnp.max(xm, axis=1, keepdims=True)
        ex = jnp.where(col < c_real, jnp.exp(agg - m), 0.0)
        lse = jnp.log(jnp.sum(ex, axis=1, keepdims=True)) + m
        o_ref[...] = (agg - lse)[:, :c_real]
    return _out_body


def _newton_rsqrt(d):
    i = plsc.bitcast(d, jnp.int32)
    y = plsc.bitcast(jnp.int32(0x5F3759DF) - (i >> 1), jnp.float32)
    for _ in range(3):
        y = y * (1.5 - 0.5 * d * y * y)
    return y


def _make_sc_kernels(n_pad, h_dim, epw, n_real):
    rps = n_pad // NS
    kc = epw // CH
    tail = epw - kc * CH
    kc_main = (kc // (2 * GRP)) * (2 * GRP)
    n_pairs = kc_main // (2 * GRP)
    rem = kc - kc_main
    rem_a = min(rem, GRP)
    rem_b = rem - rem_a

    mesh = plsc.VectorSubcoreMesh(core_axis_name="c", subcore_axis_name="s")
    out_t = jax.ShapeDtypeStruct((NC, n_pad, h_dim), jnp.float32)
    cparams = pltpu.CompilerParams(use_tc_tiling_on_sc=False,
                                   needs_layout_passes=False)

    @functools.partial(
        pl.kernel, mesh=mesh, out_type=out_t, compiler_params=cparams,
        scratch_types=[
            pltpu.VMEM_SHARED((n_pad, h_dim), jnp.float32),
            pltpu.VMEM((epw,), jnp.int32),
            pltpu.VMEM((CH, h_dim), jnp.float32),
            pltpu.SemaphoreType.DMA,
        ])
    def deg_kernel(ei_hbm, ones_hbm, zeros_hbm, out_hbm, acc, didx_v, ones_v, sem):
        c = lax.axis_index("c")
        s = lax.axis_index("s")
        wid = c * NS + s
        pltpu.sync_copy(zeros_hbm.at[pl.ds(s * rps, rps)],
                        acc.at[pl.ds(s * rps, rps)])
        pltpu.sync_copy(ei_hbm.at[1, pl.ds(wid * epw, epw)], didx_v)
        pltpu.sync_copy(ones_hbm, ones_v)
        plsc.subcore_barrier()

        def body(t, carry):
            for i in range(GRP):
                pltpu.async_copy(
                    ones_v, acc.at[didx_v.at[pl.ds((t * GRP + i) * CH, CH)]],
                    sem, add=True)
            for i in range(GRP):
                pltpu.make_async_copy(
                    ones_v, acc.at[didx_v.at[pl.ds((t * GRP + i) * CH, CH)]],
                    sem).wait()
            return carry

        lax.fori_loop(0, kc // GRP, body, 0)
        for j in range(kc - (kc // GRP) * GRP):
            jj = (kc // GRP) * GRP + j
            pltpu.async_copy(ones_v, acc.at[didx_v.at[pl.ds(jj * CH, CH)]],
                             sem, add=True)
        for j in range(kc - (kc // GRP) * GRP):
            jj = (kc // GRP) * GRP + j
            pltpu.make_async_copy(ones_v,
                                  acc.at[didx_v.at[pl.ds(jj * CH, CH)]],
                                  sem).wait()
        if tail:
            pltpu.sync_copy(ones_v.at[pl.ds(0, tail)],
                            acc.at[didx_v.at[pl.ds(kc * CH, tail)]], add=True)
        plsc.subcore_barrier()
        pltpu.sync_copy(acc.at[pl.ds(s * rps, rps)],
                        out_hbm.at[c, pl.ds(s * rps, rps)])

    def _pipeline(acc, tab_s, sidx_v, didx_v, rows_a, rows_b, sem_a, sem_b):
        def fire(base, cnt, buf, sem):
            for i in range(cnt):
                pltpu.async_copy(
                    tab_s.at[sidx_v.at[pl.ds((base + i) * CH, CH)]],
                    buf.at[i], sem)

        def drain(base, cnt, buf, sem):
            for i in range(cnt):
                pltpu.make_async_copy(
                    tab_s.at[sidx_v.at[pl.ds((base + i) * CH, CH)]],
                    buf.at[i], sem).wait()

        def scatter(base, cnt, buf):
            for i in range(cnt):
                pltpu.sync_copy(
                    buf.at[i],
                    acc.at[didx_v.at[pl.ds((base + i) * CH, CH)]], add=True)

        fire(0, GRP, rows_a, sem_a)

        def body(t, carry):
            b0 = 2 * t * GRP
            fire(b0 + GRP, GRP, rows_b, sem_b)
            drain(b0, GRP, rows_a, sem_a)
            scatter(b0, GRP, rows_a)

            @pl.when(t < n_pairs - 1)
            def _():
                fire(b0 + 2 * GRP, GRP, rows_a, sem_a)

            drain(b0 + GRP, GRP, rows_b, sem_b)
            scatter(b0 + GRP, GRP, rows_b)
            return carry

        lax.fori_loop(0, n_pairs, body, 0)

        if rem_a:
            fire(kc_main, rem_a, rows_a, sem_a)
        if rem_b:
            fire(kc_main + rem_a, rem_b, rows_b, sem_b)
        if rem_a:
            drain(kc_main, rem_a, rows_a, sem_a)
            scatter(kc_main, rem_a, rows_a)
        if rem_b:
            drain(kc_main + rem_a, rem_b, rows_b, sem_b)
            scatter(kc_main + rem_a, rem_b, rows_b)
        if tail:
            tsl = pl.ds(kc * CH, tail)
            pltpu.async_copy(tab_s.at[sidx_v.at[tsl]],
                             rows_a.at[0, pl.ds(0, tail)], sem_a).wait()
            pltpu.sync_copy(rows_a.at[0, pl.ds(0, tail)],
                            acc.at[didx_v.at[tsl]], add=True)

    gs_scratch = [
        pltpu.VMEM_SHARED((n_pad, h_dim), jnp.float32),
        pltpu.VMEM_SHARED((n_pad, h_dim), jnp.float32),
        pltpu.VMEM((epw,), jnp.int32),
        pltpu.VMEM((epw,), jnp.int32),
        pltpu.VMEM((GRP, CH, h_dim), jnp.float32),
        pltpu.VMEM((GRP, CH, h_dim), jnp.float32),
        pltpu.VMEM((rps, h_dim), jnp.float32),
        pltpu.VMEM((rps, h_dim), jnp.float32),
        pltpu.VMEM((rps, h_dim), jnp.float32),
        pltpu.VMEM((rps, h_dim), jnp.float32),
        pltpu.SemaphoreType.DMA,
        pltpu.SemaphoreType.DMA,
    ]

    @functools.partial(
        pl.kernel, mesh=mesh, compiler_params=cparams,
        out_type=[out_t,
                  jax.ShapeDtypeStruct((n_pad, h_dim), jnp.float32),
                  jax.ShapeDtypeStruct((n_pad, h_dim), jnp.float32)],
        scratch_types=gs_scratch)
    def gs1_kernel(h_hbm, degp_hbm, ei_hbm, zeros_hbm,
                   sp_out, hp_out, dinv_out,
                   acc, tab_s, sidx_v, didx_v, rows_a, rows_b,
                   h_v, d0_v, d1_v, dinv_v, sem_a, sem_b):
        c = lax.axis_index("c")
        s = lax.axis_index("s")
        wid = c * NS + s
        sl = pl.ds(s * rps, rps)
        pltpu.sync_copy(zeros_hbm.at[sl], acc.at[sl])
        pltpu.sync_copy(h_hbm.at[sl], h_v)
        pltpu.sync_copy(degp_hbm.at[0, sl], d0_v)
        pltpu.sync_copy(degp_hbm.at[1, sl], d1_v)
        pltpu.sync_copy(ei_hbm.at[0, pl.ds(wid * epw, epw)], sidx_v)
        pltpu.sync_copy(ei_hbm.at[1, pl.ds(wid * epw, epw)], didx_v)

        def row(r, carry):
            d = d0_v[r] + d1_v[r] + 1.0
            y = _newton_rsqrt(d)
            dinv_v[r] = y
            h_v[r] = h_v[r] * y
            return carry

        lax.fori_loop(0, rps, row, 0)
        pltpu.sync_copy(h_v, tab_s.at[sl])

        @pl.when(c == 0)
        def _():
            pltpu.sync_copy(h_v, hp_out.at[sl])
            pltpu.sync_copy(dinv_v, dinv_out.at[sl])

        plsc.subcore_barrier()
        _pipeline(acc, tab_s, sidx_v, didx_v, rows_a, rows_b, sem_a, sem_b)
        plsc.subcore_barrier()
        pltpu.sync_copy(acc.at[sl], sp_out.at[c, sl])

    @functools.partial(
        pl.kernel, mesh=mesh, compiler_params=cparams,
        out_type=[out_t,
                  jax.ShapeDtypeStruct((n_pad, h_dim), jnp.float32)],
        scratch_types=gs_scratch + [pltpu.VMEM((1, h_dim), jnp.float32)])
    def gs2_kernel(sp_hbm, hp_hbm, dinv_hbm, b1_hbm, ei_hbm,
                   zeros_hbm, s2_out, g_out,
                   acc, tab_s, sidx_v, didx_v, rows_a, rows_b,
                   sp0_v, sp1_v, hp_v, dinv_v, sem_a, sem_b, b1_v):
        c = lax.axis_index("c")
        s = lax.axis_index("s")
        wid = c * NS + s
        sl = pl.ds(s * rps, rps)
        pltpu.sync_copy(zeros_hbm.at[sl], acc.at[sl])
        pltpu.sync_copy(sp_hbm.at[0, sl], sp0_v)
        pltpu.sync_copy(sp_hbm.at[1, sl], sp1_v)
        pltpu.sync_copy(hp_hbm.at[sl], hp_v)
        pltpu.sync_copy(dinv_hbm.at[sl], dinv_v)
        pltpu.sync_copy(b1_hbm, b1_v)
        pltpu.sync_copy(ei_hbm.at[0, pl.ds(wid * epw, epw)], sidx_v)
        pltpu.sync_copy(ei_hbm.at[1, pl.ds(wid * epw, epw)], didx_v)

        def row(r, carry):
            y = dinv_v[r]
            a = y * (sp0_v[r] + sp1_v[r] + hp_v[r]) + b1_v[0]
            g = jnp.maximum(a, 0.0) * y
            hp_v[r] = jnp.where(s * rps + r < n_real, g, 0.0)
            return carry

        lax.fori_loop(0, rps, row, 0)
        pltpu.sync_copy(hp_v, tab_s.at[sl])

        @pl.when(c == 0)
        def _():
            pltpu.sync_copy(hp_v, g_out.at[sl])

        plsc.subcore_barrier()
        _pipeline(acc, tab_s, sidx_v, didx_v, rows_a, rows_b, sem_a, sem_b)
        plsc.subcore_barrier()
        pltpu.sync_copy(acc.at[sl], s2_out.at[c, sl])

    return deg_kernel, gs1_kernel, gs2_kernel


def kernel(x1, edge_index1, x2, edge_index2, W1, b1, W_end, b_end,
           skip_connection):
    del x1, edge_index1, skip_connection
    n, d = x2.shape
    h_dim = W1.shape[1]
    c_dim = W_end.shape[1]
    e = edge_index2.shape[1]
    f32 = jnp.float32

    n_pad = ((n + 127) // 128) * 128
    epw = e // (NC * NS)

    zeros_tab = jnp.zeros((n_pad, h_dim), f32)
    ones_rows = jnp.ones((CH, h_dim), f32)
    w_end_p = jnp.zeros((h_dim, h_dim), f32).at[:, :c_dim].set(W_end)
    b1_row = b1.reshape(1, h_dim)
    be_row = jnp.zeros((1, h_dim), f32).at[0, :c_dim].set(b_end)

    deg_kernel, gs1_kernel, gs2_kernel = _make_sc_kernels(
        n_pad, h_dim, epw, n)

    blk = 128
    grid = n_pad // blk

    h = pl.pallas_call(
        _make_mm_body(n, blk),
        grid=(grid,),
        in_specs=[pl.BlockSpec((blk, d), lambda i: (i, 0)),
                  pl.BlockSpec((d, h_dim), lambda i: (0, 0))],
        out_specs=pl.BlockSpec((blk, h_dim), lambda i: (i, 0)),
        out_shape=jax.ShapeDtypeStruct((n_pad, h_dim), f32))(x2, W1)

    degp = deg_kernel(edge_index2, ones_rows, zeros_tab)

    sp, hp, dinv = gs1_kernel(h, degp, edge_index2, zeros_tab)

    s2p, g = gs2_kernel(sp, hp, dinv, b1_row, edge_index2, zeros_tab)

    out = pl.pallas_call(
        _make_out_body(c_dim),
        grid=(grid,),
        in_specs=[pl.BlockSpec((NC, blk, h_dim), lambda i: (0, i, 0)),
                  pl.BlockSpec((blk, h_dim), lambda i: (i, 0)),
                  pl.BlockSpec((blk, h_dim), lambda i: (i, 0)),
                  pl.BlockSpec((h_dim, h_dim), lambda i: (0, 0)),
                  pl.BlockSpec((1, h_dim), lambda i: (0, 0))],
        out_specs=pl.BlockSpec((blk, c_dim), lambda i: (i, 0)),
        out_shape=jax.ShapeDtypeStruct((n, c_dim), f32))(
            s2p, g, dinv, w_end_p, be_row)

    return out


# speedup vs baseline: 1.5736x; 1.5736x over previous
import functools

import jax
import jax.numpy as jnp
from jax import lax
from jax.experimental import pallas as pl
from jax.experimental.pallas import tpu as pltpu
from jax.experimental.pallas import tpu_sc as plsc

NC = 2
NS = 16
CH = 128
GRP = 8


def _make_mm_body(n_real, blk):
    def _mm_body(x_ref, w_ref, o_ref):
        i = pl.program_id(0)
        h = jnp.dot(x_ref[...], w_ref[...],
                    preferred_element_type=jnp.float32)
        rows = i * blk + lax.broadcasted_iota(jnp.int32, h.shape, 0)
        o_ref[...] = jnp.where(rows < n_real, h, 0.0)
    return _mm_body


def _make_out_body(c_real):
    def _out_body(s2_ref, g_ref, dinv_ref, w_ref, be_ref, o_ref):
        pre = dinv_ref[...] * (s2_ref[0] + s2_ref[1] + g_ref[...])
        agg = jnp.dot(pre, w_ref[...],
                      preferred_element_type=jnp.float32) + be_ref[...]
        col = lax.broadcasted_iota(jnp.int32, agg.shape, 1)
        xm = jnp.where(col < c_real, agg, jnp.float32(-1e30))
        m = jnp.max(xm, axis=1, keepdims=True)
        ex = jnp.where(col < c_real, jnp.exp(agg - m), 0.0)
        lse = jnp.log(jnp.sum(ex, axis=1, keepdims=True)) + m
        o_ref[...] = (agg - lse)[:, :c_real]
    return _out_body


def _newton_rsqrt(d):
    i = plsc.bitcast(d, jnp.int32)
    y = plsc.bitcast(jnp.int32(0x5F3759DF) - (i >> 1), jnp.float32)
    for _ in range(3):
        y = y * (1.5 - 0.5 * d * y * y)
    return y


def _make_sc_kernels(n_pad, h_dim, epw, n_real):
    rps = n_pad // NS
    kc = epw // CH
    tail = epw - kc * CH
    kc_main = (kc // (2 * GRP)) * (2 * GRP)
    n_pairs = kc_main // (2 * GRP)
    rem = kc - kc_main
    rem_a = min(rem, GRP)
    rem_b = rem - rem_a

    mesh = plsc.VectorSubcoreMesh(core_axis_name="c", subcore_axis_name="s")
    out_t = jax.ShapeDtypeStruct((NC, n_pad, h_dim), jnp.float32)
    cparams = pltpu.CompilerParams(use_tc_tiling_on_sc=False,
                                   needs_layout_passes=False)

    @functools.partial(
        pl.kernel, mesh=mesh, out_type=out_t, compiler_params=cparams,
        scratch_types=[
            pltpu.VMEM_SHARED((n_pad, h_dim), jnp.float32),
            pltpu.VMEM((epw,), jnp.int32),
            pltpu.VMEM((CH, h_dim), jnp.float32),
            pltpu.SemaphoreType.DMA,
        ])
    def deg_kernel(ei_hbm, ones_hbm, zeros_hbm, out_hbm, acc, didx_v, ones_v, sem):
        c = lax.axis_index("c")
        s = lax.axis_index("s")
        wid = c * NS + s
        pltpu.sync_copy(zeros_hbm.at[pl.ds(s * rps, rps)],
                        acc.at[pl.ds(s * rps, rps)])
        pltpu.sync_copy(ei_hbm.at[1, pl.ds(wid * epw, epw)], didx_v)
        pltpu.sync_copy(ones_hbm, ones_v)
        plsc.subcore_barrier()

        def body(t, carry):
            for i in range(GRP):
                pltpu.async_copy(
                    ones_v, acc.at[didx_v.at[pl.ds((t * GRP + i) * CH, CH)]],
                    sem, add=True)
            for i in range(GRP):
                pltpu.make_async_copy(
                    ones_v, acc.at[didx_v.at[pl.ds((t * GRP + i) * CH, CH)]],
                    sem).wait()
            return carry

        lax.fori_loop(0, kc // GRP, body, 0)
        for j in range(kc - (kc // GRP) * GRP):
            jj = (kc // GRP) * GRP + j
            pltpu.async_copy(ones_v, acc.at[didx_v.at[pl.ds(jj * CH, CH)]],
                             sem, add=True)
        for j in range(kc - (kc // GRP) * GRP):
            jj = (kc // GRP) * GRP + j
            pltpu.make_async_copy(ones_v,
                                  acc.at[didx_v.at[pl.ds(jj * CH, CH)]],
                                  sem).wait()
        if tail:
            pltpu.sync_copy(ones_v.at[pl.ds(0, tail)],
                            acc.at[didx_v.at[pl.ds(kc * CH, tail)]], add=True)
        plsc.subcore_barrier()
        pltpu.sync_copy(acc.at[pl.ds(s * rps, rps)],
                        out_hbm.at[c, pl.ds(s * rps, rps)])

    def _pipeline(acc, tab_s, sidx_v, didx_v, rows_a, rows_b, sem_a, sem_b):
        def fire(base, cnt, buf, sem):
            for i in range(cnt):
                pltpu.async_copy(
                    tab_s.at[sidx_v.at[pl.ds((base + i) * CH, CH)]],
                    buf.at[i], sem)

        def drain(base, cnt, buf, sem):
            for i in range(cnt):
                pltpu.make_async_copy(
                    tab_s.at[sidx_v.at[pl.ds((base + i) * CH, CH)]],
                    buf.at[i], sem).wait()

        def scatter(base, cnt, buf):
            for i in range(cnt):
                pltpu.sync_copy(
                    buf.at[i],
                    acc.at[didx_v.at[pl.ds((base + i) * CH, CH)]], add=True)

        fire(0, GRP, rows_a, sem_a)

        def body(t, carry):
            b0 = 2 * t * GRP
            fire(b0 + GRP, GRP, rows_b, sem_b)
            drain(b0, GRP, rows_a, sem_a)
            scatter(b0, GRP, rows_a)

            @pl.when(t < n_pairs - 1)
            def _():
                fire(b0 + 2 * GRP, GRP, rows_a, sem_a)

            drain(b0 + GRP, GRP, rows_b, sem_b)
            scatter(b0 + GRP, GRP, rows_b)
            return carry

        lax.fori_loop(0, n_pairs, body, 0)

        if rem_a:
            fire(kc_main, rem_a, rows_a, sem_a)
        if rem_b:
            fire(kc_main + rem_a, rem_b, rows_b, sem_b)
        if rem_a:
            drain(kc_main, rem_a, rows_a, sem_a)
            scatter(kc_main, rem_a, rows_a)
        if rem_b:
            drain(kc_main + rem_a, rem_b, rows_b, sem_b)
            scatter(kc_main + rem_a, rem_b, rows_b)
        if tail:
            tsl = pl.ds(kc * CH, tail)
            pltpu.async_copy(tab_s.at[sidx_v.at[tsl]],
                             rows_a.at[0, pl.ds(0, tail)], sem_a).wait()
            pltpu.sync_copy(rows_a.at[0, pl.ds(0, tail)],
                            acc.at[didx_v.at[tsl]], add=True)

    gs_scratch = [
        pltpu.VMEM_SHARED((n_pad, h_dim), jnp.float32),
        pltpu.VMEM_SHARED((n_pad, h_dim), jnp.float32),
        pltpu.VMEM((epw,), jnp.int32),
        pltpu.VMEM((epw,), jnp.int32),
        pltpu.VMEM((GRP, CH, h_dim), jnp.float32),
        pltpu.VMEM((GRP, CH, h_dim), jnp.float32),
        pltpu.VMEM((rps, h_dim), jnp.float32),
        pltpu.VMEM((rps, h_dim), jnp.float32),
        pltpu.VMEM((rps, h_dim), jnp.float32),
        pltpu.VMEM((rps, h_dim), jnp.float32),
        pltpu.SemaphoreType.DMA,
        pltpu.SemaphoreType.DMA,
    ]

    @functools.partial(
        pl.kernel, mesh=mesh, compiler_params=cparams,
        out_type=[out_t,
                  jax.ShapeDtypeStruct((n_pad, h_dim), jnp.float32),
                  jax.ShapeDtypeStruct((n_pad, h_dim), jnp.float32)],
        scratch_types=gs_scratch)
    def gs1_kernel(h_hbm, degp_hbm, ei_hbm, zeros_hbm,
                   sp_out, hp_out, dinv_out,
                   acc, tab_s, sidx_v, didx_v, rows_a, rows_b,
                   h_v, d0_v, d1_v, dinv_v, sem_a, sem_b):
        c = lax.axis_index("c")
        s = lax.axis_index("s")
        wid = c * NS + s
        sl = pl.ds(s * rps, rps)
        pltpu.sync_copy(zeros_hbm.at[sl], acc.at[sl])
        pltpu.sync_copy(h_hbm.at[sl], h_v)
        pltpu.sync_copy(degp_hbm.at[0, sl], d0_v)
        pltpu.sync_copy(degp_hbm.at[1, sl], d1_v)
        pltpu.sync_copy(ei_hbm.at[0, pl.ds(wid * epw, epw)], sidx_v)
        pltpu.sync_copy(ei_hbm.at[1, pl.ds(wid * epw, epw)], didx_v)

        def row(r, carry):
            d = d0_v[r] + d1_v[r] + 1.0
            y = _newton_rsqrt(d)
            dinv_v[r] = y
            h_v[r] = h_v[r] * y
            return carry

        lax.fori_loop(0, rps, row, 0)
        pltpu.sync_copy(h_v, tab_s.at[sl])

        @pl.when(c == 0)
        def _():
            pltpu.sync_copy(h_v, hp_out.at[sl])
            pltpu.sync_copy(dinv_v, dinv_out.at[sl])

        plsc.subcore_barrier()
        _pipeline(acc, tab_s, sidx_v, didx_v, rows_a, rows_b, sem_a, sem_b)
        plsc.subcore_barrier()
        pltpu.sync_copy(acc.at[sl], sp_out.at[c, sl])

    @functools.partial(
        pl.kernel, mesh=mesh, compiler_params=cparams,
        out_type=[out_t,
                  jax.ShapeDtypeStruct((n_pad, h_dim), jnp.float32)],
        scratch_types=gs_scratch + [pltpu.VMEM((1, h_dim), jnp.float32)])
    def gs2_kernel(sp_hbm, hp_hbm, dinv_hbm, b1_hbm, ei_hbm,
                   zeros_hbm, s2_out, g_out,
                   acc, tab_s, sidx_v, didx_v, rows_a, rows_b,
                   sp0_v, sp1_v, hp_v, dinv_v, sem_a, sem_b, b1_v):
        c = lax.axis_index("c")
        s = lax.axis_index("s")
        wid = c * NS + s
        sl = pl.ds(s * rps, rps)
        pltpu.sync_copy(zeros_hbm.at[sl], acc.at[sl])
        pltpu.sync_copy(sp_hbm.at[0, sl], sp0_v)
        pltpu.sync_copy(sp_hbm.at[1, sl], sp1_v)
        pltpu.sync_copy(hp_hbm.at[sl], hp_v)
        pltpu.sync_copy(dinv_hbm.at[sl], dinv_v)
        pltpu.sync_copy(b1_hbm, b1_v)
        pltpu.sync_copy(ei_hbm.at[0, pl.ds(wid * epw, epw)], sidx_v)
        pltpu.sync_copy(ei_hbm.at[1, pl.ds(wid * epw, epw)], didx_v)

        def row(r, carry):
            y = dinv_v[r]
            a = y * (sp0_v[r] + sp1_v[r] + hp_v[r]) + b1_v[0]
            g = jnp.maximum(a, 0.0) * y
            hp_v[r] = jnp.where(s * rps + r < n_real, g, 0.0)
            return carry

        lax.fori_loop(0, rps, row, 0)
        pltpu.sync_copy(hp_v, tab_s.at[sl])

        @pl.when(c == 0)
        def _():
            pltpu.sync_copy(hp_v, g_out.at[sl])

        plsc.subcore_barrier()
        _pipeline(acc, tab_s, sidx_v, didx_v, rows_a, rows_b, sem_a, sem_b)
        plsc.subcore_barrier()
        pltpu.sync_copy(acc.at[sl], s2_out.at[c, sl])

    return deg_kernel, gs1_kernel, gs2_kernel


def kernel(x1, edge_index1, x2, edge_index2, W1, b1, W_end, b_end,
           skip_connection):
    del x1, edge_index1, skip_connection
    n, d = x2.shape
    h_dim = W1.shape[1]
    c_dim = W_end.shape[1]
    e = edge_index2.shape[1]
    f32 = jnp.float32

    n_pad = ((n + 127) // 128) * 128
    epw = e // (NC * NS)

    zeros_tab = jnp.zeros((n_pad, h_dim), f32)
    ones_rows = jnp.ones((CH, h_dim), f32)
    w_end_p = jnp.zeros((h_dim, h_dim), f32).at[:, :c_dim].set(W_end)
    b1_row = b1.reshape(1, h_dim)
    be_row = jnp.zeros((1, h_dim), f32).at[0, :c_dim].set(b_end)

    deg_kernel, gs1_kernel, gs2_kernel = _make_sc_kernels(
        n_pad, h_dim, epw, n)

    blk = n_pad // 4
    grid = n_pad // blk

    h = pl.pallas_call(
        _make_mm_body(n, blk),
        grid=(grid,),
        in_specs=[pl.BlockSpec((blk, d), lambda i: (i, 0)),
                  pl.BlockSpec((d, h_dim), lambda i: (0, 0))],
        out_specs=pl.BlockSpec((blk, h_dim), lambda i: (i, 0)),
        out_shape=jax.ShapeDtypeStruct((n_pad, h_dim), f32))(x2, W1)

    degp = deg_kernel(edge_index2, ones_rows, zeros_tab)

    sp, hp, dinv = gs1_kernel(h, degp, edge_index2, zeros_tab)

    s2p, g = gs2_kernel(sp, hp, dinv, b1_row, edge_index2, zeros_tab)

    out = pl.pallas_call(
        _make_out_body(c_dim),
        grid=(grid,),
        in_specs=[pl.BlockSpec((NC, blk, h_dim), lambda i: (0, i, 0)),
                  pl.BlockSpec((blk, h_dim), lambda i: (i, 0)),
                  pl.BlockSpec((blk, h_dim), lambda i: (i, 0)),
                  pl.BlockSpec((h_dim, h_dim), lambda i: (0, 0)),
                  pl.BlockSpec((1, h_dim), lambda i: (0, 0))],
        out_specs=pl.BlockSpec((blk, c_dim), lambda i: (i, 0)),
        out_shape=jax.ShapeDtypeStruct((n, c_dim), f32))(
            s2p, g, dinv, w_end_p, be_row)

    return out


# speedup vs baseline: 1.5996x; 1.0165x over previous
import functools

import jax
import jax.numpy as jnp
from jax import lax
from jax.experimental import pallas as pl
from jax.experimental.pallas import tpu as pltpu
from jax.experimental.pallas import tpu_sc as plsc

NC = 2
NS = 16
CH = 128
GRP = 8


def _make_mm_body(n_real, blk):
    def _mm_body(x_ref, w_ref, o_ref):
        i = pl.program_id(0)
        h = jnp.dot(x_ref[...], w_ref[...],
                    preferred_element_type=jnp.float32)
        rows = i * blk + lax.broadcasted_iota(jnp.int32, h.shape, 0)
        o_ref[...] = jnp.where(rows < n_real, h, 0.0)
    return _mm_body


def _make_out_body(c_real):
    def _out_body(s2_ref, g_ref, dinv_ref, w_ref, be_ref, o_ref):
        pre = dinv_ref[...] * (s2_ref[0] + s2_ref[1] + g_ref[...])
        agg = jnp.dot(pre, w_ref[...],
                      preferred_element_type=jnp.float32) + be_ref[...]
        col = lax.broadcasted_iota(jnp.int32, agg.shape, 1)
        xm = jnp.where(col < c_real, agg, jnp.float32(-1e30))
        m = jnp.max(xm, axis=1, keepdims=True)
        ex = jnp.where(col < c_real, jnp.exp(agg - m), 0.0)
        lse = jnp.log(jnp.sum(ex, axis=1, keepdims=True)) + m
        o_ref[...] = (agg - lse)[:, :c_real]
    return _out_body


def _newton_rsqrt(d):
    i = plsc.bitcast(d, jnp.int32)
    y = plsc.bitcast(jnp.int32(0x5F3759DF) - (i >> 1), jnp.float32)
    for _ in range(3):
        y = y * (1.5 - 0.5 * d * y * y)
    return y


def _make_sc_kernels(n_pad, h_dim, epw, n_real):
    rps = n_pad // NS
    kc = epw // CH
    tail = epw - kc * CH
    kc_main = (kc // (2 * GRP)) * (2 * GRP)
    n_pairs = kc_main // (2 * GRP)
    rem = kc - kc_main
    rem_a = min(rem, GRP)
    rem_b = rem - rem_a

    mesh = plsc.VectorSubcoreMesh(core_axis_name="c", subcore_axis_name="s")
    out_t = jax.ShapeDtypeStruct((NC, n_pad, h_dim), jnp.float32)
    cparams = pltpu.CompilerParams(use_tc_tiling_on_sc=False,
                                   needs_layout_passes=False)

    @functools.partial(
        pl.kernel, mesh=mesh, out_type=out_t, compiler_params=cparams,
        scratch_types=[
            pltpu.VMEM_SHARED((n_pad, h_dim), jnp.float32),
            pltpu.VMEM((epw,), jnp.int32),
            pltpu.VMEM((CH, h_dim), jnp.float32),
            pltpu.SemaphoreType.DMA,
        ])
    def deg_kernel(ei_hbm, ones_hbm, zeros_hbm, out_hbm, acc, didx_v, ones_v, sem):
        c = lax.axis_index("c")
        s = lax.axis_index("s")
        wid = c * NS + s
        pltpu.sync_copy(zeros_hbm.at[pl.ds(s * rps, rps)],
                        acc.at[pl.ds(s * rps, rps)])
        pltpu.sync_copy(ei_hbm.at[1, pl.ds(wid * epw, epw)], didx_v)
        pltpu.sync_copy(ones_hbm, ones_v)
        plsc.subcore_barrier()

        def body(t, carry):
            for i in range(GRP):
                pltpu.async_copy(
                    ones_v, acc.at[didx_v.at[pl.ds((t * GRP + i) * CH, CH)]],
                    sem, add=True)
            for i in range(GRP):
                pltpu.make_async_copy(
                    ones_v, acc.at[didx_v.at[pl.ds((t * GRP + i) * CH, CH)]],
                    sem).wait()
            return carry

        lax.fori_loop(0, kc // GRP, body, 0)
        for j in range(kc - (kc // GRP) * GRP):
            jj = (kc // GRP) * GRP + j
            pltpu.async_copy(ones_v, acc.at[didx_v.at[pl.ds(jj * CH, CH)]],
                             sem, add=True)
        for j in range(kc - (kc // GRP) * GRP):
            jj = (kc // GRP) * GRP + j
            pltpu.make_async_copy(ones_v,
                                  acc.at[didx_v.at[pl.ds(jj * CH, CH)]],
                                  sem).wait()
        if tail:
            pltpu.sync_copy(ones_v.at[pl.ds(0, tail)],
                            acc.at[didx_v.at[pl.ds(kc * CH, tail)]], add=True)
        plsc.subcore_barrier()
        pltpu.sync_copy(acc.at[pl.ds(s * rps, rps)],
                        out_hbm.at[c, pl.ds(s * rps, rps)])

    def _pipeline(acc, tab_s, sidx_v, didx_v, rows_a, rows_b, sem_a, sem_b):
        def fire(base, cnt, buf, sem):
            for i in range(cnt):
                pltpu.async_copy(
                    tab_s.at[sidx_v.at[pl.ds((base + i) * CH, CH)]],
                    buf.at[i], sem)

        def drain(base, cnt, buf, sem):
            for i in range(cnt):
                pltpu.make_async_copy(
                    tab_s.at[sidx_v.at[pl.ds((base + i) * CH, CH)]],
                    buf.at[i], sem).wait()

        def scatter(base, cnt, buf):
            for i in range(cnt):
                pltpu.sync_copy(
                    buf.at[i],
                    acc.at[didx_v.at[pl.ds((base + i) * CH, CH)]], add=True)

        fire(0, GRP, rows_a, sem_a)

        def body(t, carry):
            b0 = 2 * t * GRP
            fire(b0 + GRP, GRP, rows_b, sem_b)
            drain(b0, GRP, rows_a, sem_a)
            scatter(b0, GRP, rows_a)

            @pl.when(t < n_pairs - 1)
            def _():
                fire(b0 + 2 * GRP, GRP, rows_a, sem_a)

            drain(b0 + GRP, GRP, rows_b, sem_b)
            scatter(b0 + GRP, GRP, rows_b)
            return carry

        lax.fori_loop(0, n_pairs, body, 0)

        if rem_a:
            fire(kc_main, rem_a, rows_a, sem_a)
        if rem_b:
            fire(kc_main + rem_a, rem_b, rows_b, sem_b)
        if rem_a:
            drain(kc_main, rem_a, rows_a, sem_a)
            scatter(kc_main, rem_a, rows_a)
        if rem_b:
            drain(kc_main + rem_a, rem_b, rows_b, sem_b)
            scatter(kc_main + rem_a, rem_b, rows_b)
        if tail:
            tsl = pl.ds(kc * CH, tail)
            pltpu.async_copy(tab_s.at[sidx_v.at[tsl]],
                             rows_a.at[0, pl.ds(0, tail)], sem_a).wait()
            pltpu.sync_copy(rows_a.at[0, pl.ds(0, tail)],
                            acc.at[didx_v.at[tsl]], add=True)

    gs_scratch = [
        pltpu.VMEM_SHARED((n_pad, h_dim), jnp.float32),
        pltpu.VMEM_SHARED((n_pad, h_dim), jnp.float32),
        pltpu.VMEM((epw,), jnp.int32),
        pltpu.VMEM((epw,), jnp.int32),
        pltpu.VMEM((GRP, CH, h_dim), jnp.float32),
        pltpu.VMEM((GRP, CH, h_dim), jnp.float32),
        pltpu.VMEM((rps, h_dim), jnp.float32),
        pltpu.VMEM((rps, h_dim), jnp.float32),
        pltpu.VMEM((rps, h_dim), jnp.float32),
        pltpu.VMEM((rps, h_dim), jnp.float32),
        pltpu.SemaphoreType.DMA,
        pltpu.SemaphoreType.DMA,
    ]

    @functools.partial(
        pl.kernel, mesh=mesh, compiler_params=cparams,
        out_type=[out_t,
                  jax.ShapeDtypeStruct((n_pad, h_dim), jnp.float32),
                  jax.ShapeDtypeStruct((n_pad, h_dim), jnp.float32)],
        scratch_types=gs_scratch)
    def gs1_kernel(h_hbm, degp_hbm, ei_hbm, zeros_hbm,
                   sp_out, hp_out, dinv_out,
                   acc, tab_s, sidx_v, didx_v, rows_a, rows_b,
                   h_v, d0_v, d1_v, dinv_v, sem_a, sem_b):
        c = lax.axis_index("c")
        s = lax.axis_index("s")
        wid = c * NS + s
        sl = pl.ds(s * rps, rps)
        pltpu.sync_copy(zeros_hbm.at[sl], acc.at[sl])
        pltpu.sync_copy(h_hbm.at[sl], h_v)
        pltpu.sync_copy(degp_hbm.at[0, sl], d0_v)
        pltpu.sync_copy(degp_hbm.at[1, sl], d1_v)
        pltpu.sync_copy(ei_hbm.at[0, pl.ds(wid * epw, epw)], sidx_v)
        pltpu.sync_copy(ei_hbm.at[1, pl.ds(wid * epw, epw)], didx_v)

        unroll = 4 if rps % 4 == 0 else 1

        def row(r0, carry):
            for u in range(unroll):
                r = r0 * unroll + u
                d = d0_v[r] + d1_v[r] + 1.0
                y = _newton_rsqrt(d)
                dinv_v[r] = y
                h_v[r] = h_v[r] * y
            return carry

        lax.fori_loop(0, rps // unroll, row, 0)
        pltpu.sync_copy(h_v, tab_s.at[sl])

        @pl.when(c == 0)
        def _():
            pltpu.sync_copy(h_v, hp_out.at[sl])
            pltpu.sync_copy(dinv_v, dinv_out.at[sl])

        plsc.subcore_barrier()
        _pipeline(acc, tab_s, sidx_v, didx_v, rows_a, rows_b, sem_a, sem_b)
        plsc.subcore_barrier()
        pltpu.sync_copy(acc.at[sl], sp_out.at[c, sl])

    @functools.partial(
        pl.kernel, mesh=mesh, compiler_params=cparams,
        out_type=[out_t,
                  jax.ShapeDtypeStruct((n_pad, h_dim), jnp.float32)],
        scratch_types=gs_scratch + [pltpu.VMEM((1, h_dim), jnp.float32)])
    def gs2_kernel(sp_hbm, hp_hbm, dinv_hbm, b1_hbm, ei_hbm,
                   zeros_hbm, s2_out, g_out,
                   acc, tab_s, sidx_v, didx_v, rows_a, rows_b,
                   sp0_v, sp1_v, hp_v, dinv_v, sem_a, sem_b, b1_v):
        c = lax.axis_index("c")
        s = lax.axis_index("s")
        wid = c * NS + s
        sl = pl.ds(s * rps, rps)
        pltpu.sync_copy(zeros_hbm.at[sl], acc.at[sl])
        pltpu.sync_copy(sp_hbm.at[0, sl], sp0_v)
        pltpu.sync_copy(sp_hbm.at[1, sl], sp1_v)
        pltpu.sync_copy(hp_hbm.at[sl], hp_v)
        pltpu.sync_copy(dinv_hbm.at[sl], dinv_v)
        pltpu.sync_copy(b1_hbm, b1_v)
        pltpu.sync_copy(ei_hbm.at[0, pl.ds(wid * epw, epw)], sidx_v)
        pltpu.sync_copy(ei_hbm.at[1, pl.ds(wid * epw, epw)], didx_v)

        unroll = 4 if rps % 4 == 0 else 1

        def row(r0, carry):
            for u in range(unroll):
                r = r0 * unroll + u
                y = dinv_v[r]
                a = y * (sp0_v[r] + sp1_v[r] + hp_v[r]) + b1_v[0]
                g = jnp.maximum(a, 0.0) * y
                hp_v[r] = jnp.where(s * rps + r < n_real, g, 0.0)
            return carry

        lax.fori_loop(0, rps // unroll, row, 0)
        pltpu.sync_copy(hp_v, tab_s.at[sl])

        @pl.when(c == 0)
        def _():
            pltpu.sync_copy(hp_v, g_out.at[sl])

        plsc.subcore_barrier()
        _pipeline(acc, tab_s, sidx_v, didx_v, rows_a, rows_b, sem_a, sem_b)
        plsc.subcore_barrier()
        pltpu.sync_copy(acc.at[sl], s2_out.at[c, sl])

    return deg_kernel, gs1_kernel, gs2_kernel


def kernel(x1, edge_index1, x2, edge_index2, W1, b1, W_end, b_end,
           skip_connection):
    del x1, edge_index1, skip_connection
    n, d = x2.shape
    h_dim = W1.shape[1]
    c_dim = W_end.shape[1]
    e = edge_index2.shape[1]
    f32 = jnp.float32

    n_pad = ((n + 127) // 128) * 128
    epw = e // (NC * NS)

    zeros_tab = jnp.zeros((n_pad, h_dim), f32)
    ones_rows = jnp.ones((CH, h_dim), f32)
    w_end_p = jnp.zeros((h_dim, h_dim), f32).at[:, :c_dim].set(W_end)
    b1_row = b1.reshape(1, h_dim)
    be_row = jnp.zeros((1, h_dim), f32).at[0, :c_dim].set(b_end)

    deg_kernel, gs1_kernel, gs2_kernel = _make_sc_kernels(
        n_pad, h_dim, epw, n)

    blk = n_pad // 4
    grid = n_pad // blk

    h = pl.pallas_call(
        _make_mm_body(n, blk),
        grid=(grid,),
        in_specs=[pl.BlockSpec((blk, d), lambda i: (i, 0)),
                  pl.BlockSpec((d, h_dim), lambda i: (0, 0))],
        out_specs=pl.BlockSpec((blk, h_dim), lambda i: (i, 0)),
        out_shape=jax.ShapeDtypeStruct((n_pad, h_dim), f32))(x2, W1)

    degp = deg_kernel(edge_index2, ones_rows, zeros_tab)

    sp, hp, dinv = gs1_kernel(h, degp, edge_index2, zeros_tab)

    s2p, g = gs2_kernel(sp, hp, dinv, b1_row, edge_index2, zeros_tab)

    out = pl.pallas_call(
        _make_out_body(c_dim),
        grid=(grid,),
        in_specs=[pl.BlockSpec((NC, blk, h_dim), lambda i: (0, i, 0)),
                  pl.BlockSpec((blk, h_dim), lambda i: (i, 0)),
                  pl.BlockSpec((blk, h_dim), lambda i: (i, 0)),
                  pl.BlockSpec((h_dim, h_dim), lambda i: (0, 0)),
                  pl.BlockSpec((1, h_dim), lambda i: (0, 0))],
        out_specs=pl.BlockSpec((blk, c_dim), lambda i: (i, 0)),
        out_shape=jax.ShapeDtypeStruct((n, c_dim), f32))(
            s2p, g, dinv, w_end_p, be_row)

    return out


# speedup vs baseline: 1.6086x; 1.0056x over previous
import functools

import jax
import jax.numpy as jnp
from jax import lax
from jax.experimental import pallas as pl
from jax.experimental.pallas import tpu as pltpu
from jax.experimental.pallas import tpu_sc as plsc

NC = 2
NS = 16
CH = 128
GRP = 8


def _make_mm_body(n_real, blk):
    def _mm_body(x_ref, w_ref, o_ref):
        i = pl.program_id(0)
        h = jnp.dot(x_ref[...], w_ref[...],
                    preferred_element_type=jnp.float32)
        rows = i * blk + lax.broadcasted_iota(jnp.int32, h.shape, 0)
        o_ref[...] = jnp.where(rows < n_real, h, 0.0)
    return _mm_body


def _make_out_body(c_real):
    def _out_body(s2_ref, g_ref, dinv_ref, w_ref, be_ref, o_ref):
        pre = dinv_ref[...] * (s2_ref[0] + s2_ref[1] + g_ref[...])
        agg = jnp.dot(pre, w_ref[...],
                      preferred_element_type=jnp.float32) + be_ref[...]
        col = lax.broadcasted_iota(jnp.int32, agg.shape, 1)
        xm = jnp.where(col < c_real, agg, jnp.float32(-1e30))
        m = jnp.max(xm, axis=1, keepdims=True)
        ex = jnp.where(col < c_real, jnp.exp(agg - m), 0.0)
        lse = jnp.log(jnp.sum(ex, axis=1, keepdims=True)) + m
        o_ref[...] = (agg - lse)[:, :c_real]
    return _out_body


def _newton_rsqrt(d):
    i = plsc.bitcast(d, jnp.int32)
    y = plsc.bitcast(jnp.int32(0x5F3759DF) - (i >> 1), jnp.float32)
    for _ in range(3):
        y = y * (1.5 - 0.5 * d * y * y)
    return y


def _make_sc_kernels(n_pad, h_dim, epw, n_real):
    rps = n_pad // NS
    kc = epw // CH
    tail = epw - kc * CH
    kc_main = (kc // (2 * GRP)) * (2 * GRP)
    n_pairs = kc_main // (2 * GRP)
    rem = kc - kc_main
    rem_a = min(rem, GRP)
    rem_b = rem - rem_a

    mesh = plsc.VectorSubcoreMesh(core_axis_name="c", subcore_axis_name="s")
    out_t = jax.ShapeDtypeStruct((NC, n_pad, h_dim), jnp.float32)
    cparams = pltpu.CompilerParams(use_tc_tiling_on_sc=False,
                                   needs_layout_passes=False)

    @functools.partial(
        pl.kernel, mesh=mesh, out_type=out_t, compiler_params=cparams,
        scratch_types=[
            pltpu.VMEM_SHARED((n_pad, h_dim), jnp.float32),
            pltpu.VMEM((epw,), jnp.int32),
            pltpu.VMEM((CH, h_dim), jnp.float32),
            pltpu.SemaphoreType.DMA,
        ])
    def deg_kernel(ei_hbm, ones_hbm, zeros_hbm, out_hbm, acc, didx_v, ones_v, sem):
        c = lax.axis_index("c")
        s = lax.axis_index("s")
        wid = c * NS + s
        pltpu.sync_copy(zeros_hbm.at[pl.ds(s * rps, rps)],
                        acc.at[pl.ds(s * rps, rps)])
        pltpu.sync_copy(ei_hbm.at[1, pl.ds(wid * epw, epw)], didx_v)
        pltpu.sync_copy(ones_hbm, ones_v)
        plsc.subcore_barrier()

        def body(t, carry):
            for i in range(GRP):
                pltpu.async_copy(
                    ones_v, acc.at[didx_v.at[pl.ds((t * GRP + i) * CH, CH)]],
                    sem, add=True)
            for i in range(GRP):
                pltpu.make_async_copy(
                    ones_v, acc.at[didx_v.at[pl.ds((t * GRP + i) * CH, CH)]],
                    sem).wait()
            return carry

        lax.fori_loop(0, kc // GRP, body, 0)
        for j in range(kc - (kc // GRP) * GRP):
            jj = (kc // GRP) * GRP + j
            pltpu.async_copy(ones_v, acc.at[didx_v.at[pl.ds(jj * CH, CH)]],
                             sem, add=True)
        for j in range(kc - (kc // GRP) * GRP):
            jj = (kc // GRP) * GRP + j
            pltpu.make_async_copy(ones_v,
                                  acc.at[didx_v.at[pl.ds(jj * CH, CH)]],
                                  sem).wait()
        if tail:
            pltpu.sync_copy(ones_v.at[pl.ds(0, tail)],
                            acc.at[didx_v.at[pl.ds(kc * CH, tail)]], add=True)
        plsc.subcore_barrier()
        pltpu.sync_copy(acc.at[pl.ds(s * rps, rps)],
                        out_hbm.at[c, pl.ds(s * rps, rps)])

    def _pipeline(acc, tab_s, sidx_v, didx_v, rows_a, rows_b,
                  sem_a, sem_b, ssem_a, ssem_b):
        def fire(base, cnt, buf, sem):
            for i in range(cnt):
                pltpu.async_copy(
                    tab_s.at[sidx_v.at[pl.ds((base + i) * CH, CH)]],
                    buf.at[i], sem)

        def drain(base, cnt, buf, sem):
            for i in range(cnt):
                pltpu.make_async_copy(
                    tab_s.at[sidx_v.at[pl.ds((base + i) * CH, CH)]],
                    buf.at[i], sem).wait()

        def fire_s(base, cnt, buf, sem):
            for i in range(cnt):
                pltpu.async_copy(
                    buf.at[i],
                    acc.at[didx_v.at[pl.ds((base + i) * CH, CH)]],
                    sem, add=True)

        def drain_s(base, cnt, buf, sem):
            for i in range(cnt):
                pltpu.make_async_copy(
                    buf.at[i],
                    acc.at[didx_v.at[pl.ds((base + i) * CH, CH)]],
                    sem).wait()

        fire(0, GRP, rows_a, sem_a)

        def body(t, carry):
            b0 = 2 * t * GRP
            fire(b0 + GRP, GRP, rows_b, sem_b)
            drain(b0, GRP, rows_a, sem_a)
            fire_s(b0, GRP, rows_a, ssem_a)
            drain(b0 + GRP, GRP, rows_b, sem_b)
            fire_s(b0 + GRP, GRP, rows_b, ssem_b)
            drain_s(b0, GRP, rows_a, ssem_a)

            @pl.when(t < n_pairs - 1)
            def _():
                fire(b0 + 2 * GRP, GRP, rows_a, sem_a)

            drain_s(b0 + GRP, GRP, rows_b, ssem_b)
            return carry

        lax.fori_loop(0, n_pairs, body, 0)

        def scatter(base, cnt, buf):
            for i in range(cnt):
                pltpu.sync_copy(
                    buf.at[i],
                    acc.at[didx_v.at[pl.ds((base + i) * CH, CH)]], add=True)

        if rem_a:
            fire(kc_main, rem_a, rows_a, sem_a)
        if rem_b:
            fire(kc_main + rem_a, rem_b, rows_b, sem_b)
        if rem_a:
            drain(kc_main, rem_a, rows_a, sem_a)
            scatter(kc_main, rem_a, rows_a)
        if rem_b:
            drain(kc_main + rem_a, rem_b, rows_b, sem_b)
            scatter(kc_main + rem_a, rem_b, rows_b)
        if tail:
            tsl = pl.ds(kc * CH, tail)
            pltpu.async_copy(tab_s.at[sidx_v.at[tsl]],
                             rows_a.at[0, pl.ds(0, tail)], sem_a).wait()
            pltpu.sync_copy(rows_a.at[0, pl.ds(0, tail)],
                            acc.at[didx_v.at[tsl]], add=True)

    gs_scratch = [
        pltpu.VMEM_SHARED((n_pad, h_dim), jnp.float32),
        pltpu.VMEM_SHARED((n_pad, h_dim), jnp.float32),
        pltpu.VMEM((epw,), jnp.int32),
        pltpu.VMEM((epw,), jnp.int32),
        pltpu.VMEM((GRP, CH, h_dim), jnp.float32),
        pltpu.VMEM((GRP, CH, h_dim), jnp.float32),
        pltpu.VMEM((rps, h_dim), jnp.float32),
        pltpu.VMEM((rps, h_dim), jnp.float32),
        pltpu.VMEM((rps, h_dim), jnp.float32),
        pltpu.VMEM((rps, h_dim), jnp.float32),
        pltpu.SemaphoreType.DMA,
        pltpu.SemaphoreType.DMA,
        pltpu.SemaphoreType.DMA,
        pltpu.SemaphoreType.DMA,
    ]

    @functools.partial(
        pl.kernel, mesh=mesh, compiler_params=cparams,
        out_type=[out_t,
                  jax.ShapeDtypeStruct((n_pad, h_dim), jnp.float32),
                  jax.ShapeDtypeStruct((n_pad, h_dim), jnp.float32)],
        scratch_types=gs_scratch)
    def gs1_kernel(h_hbm, degp_hbm, ei_hbm, zeros_hbm,
                   sp_out, hp_out, dinv_out,
                   acc, tab_s, sidx_v, didx_v, rows_a, rows_b,
                   h_v, d0_v, d1_v, dinv_v, sem_a, sem_b, ssem_a, ssem_b):
        c = lax.axis_index("c")
        s = lax.axis_index("s")
        wid = c * NS + s
        sl = pl.ds(s * rps, rps)
        pltpu.sync_copy(zeros_hbm.at[sl], acc.at[sl])
        pltpu.sync_copy(h_hbm.at[sl], h_v)
        pltpu.sync_copy(degp_hbm.at[0, sl], d0_v)
        pltpu.sync_copy(degp_hbm.at[1, sl], d1_v)
        pltpu.sync_copy(ei_hbm.at[0, pl.ds(wid * epw, epw)], sidx_v)
        pltpu.sync_copy(ei_hbm.at[1, pl.ds(wid * epw, epw)], didx_v)

        unroll = 4 if rps % 4 == 0 else 1

        def row(r0, carry):
            for u in range(unroll):
                r = r0 * unroll + u
                d = d0_v[r] + d1_v[r] + 1.0
                y = _newton_rsqrt(d)
                dinv_v[r] = y
                h_v[r] = h_v[r] * y
            return carry

        lax.fori_loop(0, rps // unroll, row, 0)
        pltpu.sync_copy(h_v, tab_s.at[sl])

        @pl.when(c == 0)
        def _():
            pltpu.sync_copy(h_v, hp_out.at[sl])
            pltpu.sync_copy(dinv_v, dinv_out.at[sl])

        plsc.subcore_barrier()
        _pipeline(acc, tab_s, sidx_v, didx_v, rows_a, rows_b,
                  sem_a, sem_b, ssem_a, ssem_b)
        plsc.subcore_barrier()
        pltpu.sync_copy(acc.at[sl], sp_out.at[c, sl])

    @functools.partial(
        pl.kernel, mesh=mesh, compiler_params=cparams,
        out_type=[out_t,
                  jax.ShapeDtypeStruct((n_pad, h_dim), jnp.float32)],
        scratch_types=gs_scratch + [pltpu.VMEM((1, h_dim), jnp.float32)])
    def gs2_kernel(sp_hbm, hp_hbm, dinv_hbm, b1_hbm, ei_hbm,
                   zeros_hbm, s2_out, g_out,
                   acc, tab_s, sidx_v, didx_v, rows_a, rows_b,
                   sp0_v, sp1_v, hp_v, dinv_v, sem_a, sem_b,
                   ssem_a, ssem_b, b1_v):
        c = lax.axis_index("c")
        s = lax.axis_index("s")
        wid = c * NS + s
        sl = pl.ds(s * rps, rps)
        pltpu.sync_copy(zeros_hbm.at[sl], acc.at[sl])
        pltpu.sync_copy(sp_hbm.at[0, sl], sp0_v)
        pltpu.sync_copy(sp_hbm.at[1, sl], sp1_v)
        pltpu.sync_copy(hp_hbm.at[sl], hp_v)
        pltpu.sync_copy(dinv_hbm.at[sl], dinv_v)
        pltpu.sync_copy(b1_hbm, b1_v)
        pltpu.sync_copy(ei_hbm.at[0, pl.ds(wid * epw, epw)], sidx_v)
        pltpu.sync_copy(ei_hbm.at[1, pl.ds(wid * epw, epw)], didx_v)

        unroll = 4 if rps % 4 == 0 else 1

        def row(r0, carry):
            for u in range(unroll):
                r = r0 * unroll + u
                y = dinv_v[r]
                a = y * (sp0_v[r] + sp1_v[r] + hp_v[r]) + b1_v[0]
                g = jnp.maximum(a, 0.0) * y
                hp_v[r] = jnp.where(s * rps + r < n_real, g, 0.0)
            return carry

        lax.fori_loop(0, rps // unroll, row, 0)
        pltpu.sync_copy(hp_v, tab_s.at[sl])

        @pl.when(c == 0)
        def _():
            pltpu.sync_copy(hp_v, g_out.at[sl])

        plsc.subcore_barrier()
        _pipeline(acc, tab_s, sidx_v, didx_v, rows_a, rows_b,
                  sem_a, sem_b, ssem_a, ssem_b)
        plsc.subcore_barrier()
        pltpu.sync_copy(acc.at[sl], s2_out.at[c, sl])

    return deg_kernel, gs1_kernel, gs2_kernel


def kernel(x1, edge_index1, x2, edge_index2, W1, b1, W_end, b_end,
           skip_connection):
    del x1, edge_index1, skip_connection
    n, d = x2.shape
    h_dim = W1.shape[1]
    c_dim = W_end.shape[1]
    e = edge_index2.shape[1]
    f32 = jnp.float32

    n_pad = ((n + 127) // 128) * 128
    epw = e // (NC * NS)

    zeros_tab = jnp.zeros((n_pad, h_dim), f32)
    ones_rows = jnp.ones((CH, h_dim), f32)
    w_end_p = jnp.zeros((h_dim, h_dim), f32).at[:, :c_dim].set(W_end)
    b1_row = b1.reshape(1, h_dim)
    be_row = jnp.zeros((1, h_dim), f32).at[0, :c_dim].set(b_end)

    deg_kernel, gs1_kernel, gs2_kernel = _make_sc_kernels(
        n_pad, h_dim, epw, n)

    blk = n_pad // 4
    grid = n_pad // blk

    h = pl.pallas_call(
        _make_mm_body(n, blk),
        grid=(grid,),
        in_specs=[pl.BlockSpec((blk, d), lambda i: (i, 0)),
                  pl.BlockSpec((d, h_dim), lambda i: (0, 0))],
        out_specs=pl.BlockSpec((blk, h_dim), lambda i: (i, 0)),
        out_shape=jax.ShapeDtypeStruct((n_pad, h_dim), f32))(x2, W1)

    degp = deg_kernel(edge_index2, ones_rows, zeros_tab)

    sp, hp, dinv = gs1_kernel(h, degp, edge_index2, zeros_tab)

    s2p, g = gs2_kernel(sp, hp, dinv, b1_row, edge_index2, zeros_tab)

    out = pl.pallas_call(
        _make_out_body(c_dim),
        grid=(grid,),
        in_specs=[pl.BlockSpec((NC, blk, h_dim), lambda i: (0, i, 0)),
                  pl.BlockSpec((blk, h_dim), lambda i: (i, 0)),
                  pl.BlockSpec((blk, h_dim), lambda i: (i, 0)),
                  pl.BlockSpec((h_dim, h_dim), lambda i: (0, 0)),
                  pl.BlockSpec((1, h_dim), lambda i: (0, 0))],
        out_specs=pl.BlockSpec((blk, c_dim), lambda i: (i, 0)),
        out_shape=jax.ShapeDtypeStruct((n, c_dim), f32))(
            s2p, g, dinv, w_end_p, be_row)

    return out
